# Initial kernel scaffold; baseline (speedup 1.0000x reference)
import functools

import jax
import jax.numpy as jnp
from jax import lax
from jax.experimental import pallas as pl
from jax.experimental.pallas import tpu as pltpu
from jax.experimental.pallas import tpu_sc as plsc

N_NODES = 10000
N_EDGES = 320000
D_FEAT = 128

NC = 2
NS = 16
NW = NC * NS
L = 16

K = 128
NB = N_EDGES // K
ROWS_PER_TILE = N_NODES // NS


def _sc_segment_sum(feats, src, dst, zagg, zdeg):
    mesh = plsc.VectorSubcoreMesh(core_axis_name="c", subcore_axis_name="s")

    @functools.partial(
        pl.kernel,
        mesh=mesh,
        out_type=(
            jax.ShapeDtypeStruct((NC, N_NODES, D_FEAT), jnp.float32),
            jax.ShapeDtypeStruct((NW, N_NODES), jnp.float32),
        ),
        scratch_types=[
            pltpu.VMEM((K,), jnp.int32),
            pltpu.VMEM((K,), jnp.int32),
            pltpu.VMEM((K, D_FEAT), jnp.float32),
            pltpu.VMEM((N_NODES,), jnp.float32),
            pltpu.VMEM_SHARED((N_NODES, D_FEAT), jnp.float32),
            pltpu.SemaphoreType.DMA,
        ],
    )
    def k(feats_hbm, src_hbm, dst_hbm, zagg_hbm, zdeg_hbm,
          agg_out, deg_out, srcv, dstv, rows, degl, aggsh, sem):
        cid = lax.axis_index("c")
        sid = lax.axis_index("s")
        wid = sid * NC + cid

        pltpu.sync_copy(zagg_hbm.at[pl.ds(sid * ROWS_PER_TILE, ROWS_PER_TILE)],
                        aggsh.at[pl.ds(sid * ROWS_PER_TILE, ROWS_PER_TILE)])
        pltpu.sync_copy(zdeg_hbm, degl)
        plsc.subcore_barrier()

        ones = jnp.ones((L,), jnp.float32)

        n_i = jnp.where(wid < NB % NW, NB // NW + 1, NB // NW)

        def body(i, carry):
            base = (wid + i * NW) * K
            pltpu.sync_copy(src_hbm.at[pl.ds(base, K)], srcv)
            pltpu.sync_copy(dst_hbm.at[pl.ds(base, K)], dstv)
            pltpu.async_copy(feats_hbm.at[srcv], rows, sem).wait()
            pltpu.sync_copy(rows, aggsh.at[dstv], add=True)
            for j in range(K // L):
                idx = dstv[pl.ds(j * L, L)]
                plsc.addupdate_scatter(degl, [idx], ones)
            return carry

        lax.fori_loop(0, n_i, body, 0)
        plsc.subcore_barrier()

        pltpu.sync_copy(aggsh.at[pl.ds(sid * ROWS_PER_TILE, ROWS_PER_TILE)],
                        agg_out.at[cid, pl.ds(sid * ROWS_PER_TILE, ROWS_PER_TILE)])
        pltpu.sync_copy(degl, deg_out.at[wid])

    return k(feats, src, dst, zagg, zdeg)


ROW_BLK = 400


def _finish_body(agg_ref, deg_ref, x_ref, wm_ref, wx_ref, b_ref, o_ref):
    agg = agg_ref[0] + agg_ref[1]
    deg = jnp.sum(deg_ref[...], axis=0)
    inv = 1.0 / jnp.maximum(deg, 1.0)
    mean = agg * inv[:, None]
    o_ref[...] = (
        jnp.dot(mean, wm_ref[...], preferred_element_type=jnp.float32)
        + jnp.dot(x_ref[...], wx_ref[...], preferred_element_type=jnp.float32)
        + b_ref[...]
    )


def _tc_finish(aggp, degp, n_feats, wm, wx, b2):
    grid = (N_NODES // ROW_BLK,)
    return pl.pallas_call(
        _finish_body,
        grid=grid,
        in_specs=[
            pl.BlockSpec((NC, ROW_BLK, D_FEAT), lambda i: (0, i, 0)),
            pl.BlockSpec((NW, ROW_BLK), lambda i: (0, i)),
            pl.BlockSpec((ROW_BLK, D_FEAT), lambda i: (i, 0)),
            pl.BlockSpec((D_FEAT, D_FEAT), lambda i: (0, 0)),
            pl.BlockSpec((D_FEAT, D_FEAT), lambda i: (0, 0)),
            pl.BlockSpec((1, D_FEAT), lambda i: (0, 0)),
        ],
        out_specs=pl.BlockSpec((ROW_BLK, D_FEAT), lambda i: (i, 0)),
        out_shape=jax.ShapeDtypeStruct((N_NODES, D_FEAT), jnp.float32),
    )(aggp, degp, n_feats, wm, wx, b2)


def kernel(n_feats, edge_index, W, b):
    src = edge_index[0]
    dst = edge_index[1]
    zagg = jnp.zeros((N_NODES, D_FEAT), jnp.float32)
    zdeg = jnp.zeros((N_NODES,), jnp.float32)
    aggp, degp = _sc_segment_sum(n_feats, src, dst, zagg, zdeg)
    wm = W[:, :D_FEAT].T
    wx = W[:, D_FEAT:].T
    b2 = b.reshape(1, D_FEAT)
    return _tc_finish(aggp, degp, n_feats, wm, wx, b2)


# speedup vs baseline: 7.3531x; 7.3531x over previous
import functools

import jax
import jax.numpy as jnp
from jax import lax
from jax.experimental import pallas as pl
from jax.experimental.pallas import tpu as pltpu
from jax.experimental.pallas import tpu_sc as plsc

N_NODES = 10000
N_EDGES = 320000
D_FEAT = 128

NC = 2
NS = 16
NW = NC * NS
L = 16

K = 128
NB = N_EDGES // K
ROW_STRIDE = 624
ROW_CHUNK = 640


def _sc_segment_sum(feats, src, dst, zagg, zdeg):
    mesh = plsc.VectorSubcoreMesh(core_axis_name="c", subcore_axis_name="s")

    @functools.partial(
        pl.kernel,
        mesh=mesh,
        out_type=(
            jax.ShapeDtypeStruct((NC, N_NODES, D_FEAT), jnp.float32),
            jax.ShapeDtypeStruct((NW * N_NODES,), jnp.float32),
        ),
        scratch_types=[
            pltpu.VMEM((K,), jnp.int32),
            pltpu.VMEM((K,), jnp.int32),
            pltpu.VMEM((K, D_FEAT), jnp.float32),
            pltpu.VMEM((N_NODES,), jnp.float32),
            pltpu.VMEM_SHARED((N_NODES, D_FEAT), jnp.float32),
            pltpu.SemaphoreType.DMA,
        ],
        compiler_params=pltpu.CompilerParams(needs_layout_passes=False),
    )
    def k(feats_hbm, src_hbm, dst_hbm, zagg_hbm, zdeg_hbm,
          agg_out, deg_out, srcv, dstv, rows, degl, aggsh, sem):
        cid = lax.axis_index("c")
        sid = lax.axis_index("s")
        wid = sid * NC + cid

        pltpu.sync_copy(zagg_hbm.at[pl.ds(sid * ROW_STRIDE, ROW_CHUNK)],
                        aggsh.at[pl.ds(sid * ROW_STRIDE, ROW_CHUNK)])
        pltpu.sync_copy(zdeg_hbm, degl)
        plsc.subcore_barrier()

        ones = jnp.ones((L,), jnp.float32)

        n_i = jnp.where(wid < NB % NW, NB // NW + 1, NB // NW)

        def body(i, carry):
            base = (wid + i * NW) * K
            pltpu.sync_copy(src_hbm.at[pl.ds(base, K)], srcv)
            pltpu.sync_copy(dst_hbm.at[pl.ds(base, K)], dstv)
            pltpu.async_copy(feats_hbm.at[srcv], rows, sem).wait()
            pltpu.sync_copy(rows, aggsh.at[dstv], add=True)
            for j in range(K // L):
                idx = dstv[pl.ds(j * L, L)]
                plsc.addupdate_scatter(degl, [idx], ones)
            return carry

        lax.fori_loop(0, n_i, body, 0)
        plsc.subcore_barrier()

        pltpu.sync_copy(aggsh.at[pl.ds(sid * ROW_STRIDE, ROW_CHUNK)],
                        agg_out.at[cid, pl.ds(sid * ROW_STRIDE, ROW_CHUNK)])
        pltpu.sync_copy(degl, deg_out.at[pl.ds(wid * N_NODES, N_NODES)])

    return k(feats, src, dst, zagg, zdeg)


ROW_BLK = 400


def _finish_body(agg_ref, deg_ref, x_ref, wm_ref, wx_ref, b_ref, o_ref):
    agg = agg_ref[0] + agg_ref[1]
    deg = jnp.sum(deg_ref[...], axis=1)
    inv = 1.0 / jnp.maximum(deg, 1.0)
    mean = agg * inv[:, None]
    o_ref[...] = (
        jnp.dot(mean, wm_ref[...], preferred_element_type=jnp.float32)
        + jnp.dot(x_ref[...], wx_ref[...], preferred_element_type=jnp.float32)
        + b_ref[...]
    )


def _tc_finish(aggp, degp, n_feats, wm, wx, b2):
    grid = (N_NODES // ROW_BLK,)
    return pl.pallas_call(
        _finish_body,
        grid=grid,
        in_specs=[
            pl.BlockSpec((NC, ROW_BLK, D_FEAT), lambda i: (0, i, 0)),
            pl.BlockSpec((ROW_BLK, NW), lambda i: (i, 0)),
            pl.BlockSpec((ROW_BLK, D_FEAT), lambda i: (i, 0)),
            pl.BlockSpec((D_FEAT, D_FEAT), lambda i: (0, 0)),
            pl.BlockSpec((D_FEAT, D_FEAT), lambda i: (0, 0)),
            pl.BlockSpec((1, D_FEAT), lambda i: (0, 0)),
        ],
        out_specs=pl.BlockSpec((ROW_BLK, D_FEAT), lambda i: (i, 0)),
        out_shape=jax.ShapeDtypeStruct((N_NODES, D_FEAT), jnp.float32),
    )(aggp, degp, n_feats, wm, wx, b2)


def kernel(n_feats, edge_index, W, b):
    src = edge_index[0]
    dst = edge_index[1]
    zagg = jnp.zeros((N_NODES, D_FEAT), jnp.float32)
    zdeg = jnp.zeros((N_NODES,), jnp.float32)
    aggp, degp = _sc_segment_sum(n_feats, src, dst, zagg, zdeg)
    degp = degp.reshape(NW, N_NODES).T
    wm = W[:, :D_FEAT].T
    wx = W[:, D_FEAT:].T
    b2 = b.reshape(1, D_FEAT)
    return _tc_finish(aggp, degp, n_feats, wm, wx, b2)
